# baseline (speedup 1.0000x reference)
import functools

import jax
import jax.numpy as jnp
from jax import lax
from jax.experimental import pallas as pl
from jax.experimental.pallas import tpu as pltpu
from jax.experimental.pallas import tpu_sc as plsc

_N_ITEMS = 1000000
_N_OUT = 1000
_D = 64
_B = 4096
_K = 16


def _make_sc_gather(B, D):
    info = plsc.get_sparse_core_info()
    NC, NS, L = info.num_cores, info.num_subcores, info.num_lanes
    NW = NC * NS
    b_per_w = B // NW
    n_chunks = b_per_w // _K
    mesh = plsc.VectorSubcoreMesh(core_axis_name="c", subcore_axis_name="s")

    @functools.partial(
        pl.kernel,
        mesh=mesh,
        out_type=jax.ShapeDtypeStruct((B, D), jnp.float32),
        compiler_params=pltpu.CompilerParams(needs_layout_passes=False),
        scratch_types=[
            pltpu.VMEM((b_per_w,), jnp.int32),
            pltpu.VMEM((_K, 8, D), jnp.float32),
            pltpu.VMEM((b_per_w, D), jnp.float32),
            pltpu.SemaphoreType.DMA,
        ],
    )
    def gather_k(table_hbm, idx_hbm, out_hbm, idx_v, ring_v, rows_v, sem):
        lane = lax.iota(jnp.int32, L)
        wid = lax.axis_index("s") * NC + lax.axis_index("c")
        base = wid * b_per_w
        pltpu.sync_copy(idx_hbm.at[pl.ds(base, b_per_w)], idx_v)

        for c in range(n_chunks):
            ivec = idx_v[pl.ds(c * _K, _K)]
            tvec = ivec >> 3
            copies = []
            for l in range(_K):
                t = jnp.sum(jnp.where(lane == l, tvec, 0))
                r0 = pl.multiple_of(t * 8, 8)
                copies.append(
                    pltpu.async_copy(
                        table_hbm.at[pl.ds(r0, 8), :], ring_v.at[l], sem
                    )
                )
            for cp in copies:
                cp.wait()

            svec = ivec & 7
            kvec = lane + c * _K

            @pl.loop(0, D)
            def _(j):
                jvec = jnp.full((L,), j, jnp.int32)
                col = plsc.load_gather(ring_v, [lane, svec, jvec])
                plsc.store_scatter(rows_v, [kvec, jvec], col)

        pltpu.sync_copy(rows_v, out_hbm.at[pl.ds(base, b_per_w)])

    return gather_k


def _dense_body(z_ref, w_ref, b_ref, o_ref):
    logits = (
        jnp.dot(z_ref[...], w_ref[...], preferred_element_type=jnp.float32)
        + b_ref[...]
    )
    m = jnp.max(logits, axis=-1, keepdims=True)
    e = jnp.exp(logits - m)
    o_ref[...] = e / jnp.sum(e, axis=-1, keepdims=True)


def _dense(z, w_t, b2, bm):
    B = z.shape[0]
    n_out = w_t.shape[1]
    return pl.pallas_call(
        _dense_body,
        grid=(B // bm,),
        in_specs=[
            pl.BlockSpec((bm, _D), lambda i: (i, 0)),
            pl.BlockSpec((_D, n_out), lambda i: (0, 0)),
            pl.BlockSpec((1, n_out), lambda i: (0, 0)),
        ],
        out_specs=pl.BlockSpec((bm, n_out), lambda i: (i, 0)),
        out_shape=jax.ShapeDtypeStruct((B, n_out), jnp.float32),
    )(z, w_t, b2)


def kernel(item_ids, emb_table, fc_w, fc_b):
    idx = item_ids.astype(jnp.int32)
    z = _make_sc_gather(_B, _D)(emb_table, idx)
    w_t = fc_w.T
    b2 = fc_b.reshape(1, _N_OUT)
    return _dense(z, w_t, b2, bm=512)


# speedup vs baseline: 4.1303x; 4.1303x over previous
import functools

import jax
import jax.numpy as jnp
from jax import lax
from jax.experimental import pallas as pl
from jax.experimental.pallas import tpu as pltpu
from jax.experimental.pallas import tpu_sc as plsc

_N_ITEMS = 1000000
_N_OUT = 1000
_D = 64
_B = 4096
_CK = 8
_CW = 128


def _make_sc_gather(B, D):
    info = plsc.get_sparse_core_info()
    NC, NS, L = info.num_cores, info.num_subcores, info.num_lanes
    NW = NC * NS
    b_per_w = B // NW
    n_chunks = b_per_w // _CK
    mesh = plsc.VectorSubcoreMesh(core_axis_name="c", subcore_axis_name="s")

    @functools.partial(
        pl.kernel,
        mesh=mesh,
        out_type=jax.ShapeDtypeStruct((D, B), jnp.float32),
        compiler_params=pltpu.CompilerParams(needs_layout_passes=False),
        scratch_types=[
            pltpu.VMEM((b_per_w + L, ), jnp.int32),
            pltpu.VMEM((_CK, D, _CW), jnp.float32),
            pltpu.VMEM((D, b_per_w), jnp.float32),
            pltpu.SemaphoreType.DMA,
        ],
    )
    def gather_k(table_hbm, idx_hbm, out_hbm, idx_v, ring_v, zt_v, sem):
        lane = lax.iota(jnp.int32, L)
        wid = lax.axis_index("s") * NC + lax.axis_index("c")
        base = wid * b_per_w
        pltpu.sync_copy(
            idx_hbm.at[pl.ds(base, b_per_w)], idx_v.at[pl.ds(0, b_per_w)]
        )

        @pl.loop(0, n_chunks)
        def _(ch):
            k0 = ch * _CK
            ivec = idx_v[pl.ds(k0, L)]
            cvec = (ivec >> 7) << 7
            lvec = ivec & (_CW - 1)

            copies = []
            lscal = []
            for j in range(_CK):
                c = jnp.sum(jnp.where(lane == j, cvec, 0))
                lscal.append(jnp.sum(jnp.where(lane == j, lvec, 0)))
                col0 = pl.multiple_of(c, _CW)
                copies.append(
                    pltpu.async_copy(
                        table_hbm.at[:, pl.ds(col0, _CW)], ring_v.at[j], sem
                    )
                )
            for cp in copies:
                cp.wait()

            for j in range(_CK):
                lj = lscal[j]
                for c4 in range(D // L):
                    rows = lane + c4 * L
                    vals = plsc.load_gather(
                        ring_v.at[j], [rows, jnp.full((L,), lj, jnp.int32)]
                    )
                    plsc.store_scatter(
                        zt_v, [rows, jnp.full((L,), k0 + j, jnp.int32)], vals
                    )

        pltpu.sync_copy(zt_v, out_hbm.at[:, pl.ds(base, b_per_w)])

    return gather_k


def _dense_body(w_ref, zt_ref, b_ref, o_ref):
    logits = (
        lax.dot_general(
            w_ref[...], zt_ref[...],
            (((1,), (0,)), ((), ())),
            preferred_element_type=jnp.float32,
        )
        + b_ref[...]
    )
    m = jnp.max(logits, axis=0, keepdims=True)
    e = jnp.exp(logits - m)
    o_ref[...] = e / jnp.sum(e, axis=0, keepdims=True)


def _dense_t(w, zt, bcol, bm):
    B = zt.shape[1]
    n_out = w.shape[0]
    return pl.pallas_call(
        _dense_body,
        grid=(B // bm,),
        in_specs=[
            pl.BlockSpec((n_out, _D), lambda i: (0, 0)),
            pl.BlockSpec((_D, bm), lambda i: (0, i)),
            pl.BlockSpec((n_out, 1), lambda i: (0, 0)),
        ],
        out_specs=pl.BlockSpec((n_out, bm), lambda i: (0, i)),
        out_shape=jax.ShapeDtypeStruct((n_out, B), jnp.float32),
    )(w, zt, bcol)


def kernel(item_ids, emb_table, fc_w, fc_b):
    idx = item_ids.astype(jnp.int32)
    table_t = emb_table.T
    zt = _make_sc_gather(_B, _D)(table_t, idx)
    bcol = fc_b.reshape(_N_OUT, 1)
    out_t = _dense_t(fc_w, zt, bcol, bm=512)
    return out_t.T
